# baseline (speedup 1.0000x reference)
import functools

import jax
import jax.numpy as jnp
from jax import lax
from jax.experimental import pallas as pl
from jax.experimental.pallas import tpu as pltpu
from jax.experimental.pallas import tpu_sc as plsc

MASK_RATIO = 0.4
MASK_VALUE = 0.0

_SC_LANES = 16


def _mask_kernel(noise_ref, mask_ref, mult_ref, *, num_remove):
    n = noise_ref[0]
    S = n.shape[-1]
    k2 = pltpu.bitcast(n, jnp.int32) * 2
    j_idx = lax.broadcasted_iota(jnp.int32, (1, S, S), 1)
    i_idx = lax.broadcasted_iota(jnp.int32, (1, S, S), 2)
    tri = (j_idx > i_idx).astype(jnp.int32)
    bj = k2[:, :, None] + tri
    greater = bj > k2[:, None, :]
    cnt = jnp.count_nonzero(greater, axis=1).astype(jnp.int32)
    remove = cnt < num_remove
    mask_ref[0] = remove.astype(jnp.float32)
    mult_ref[0] = jnp.where(remove, jnp.float32(MASK_VALUE), jnp.float32(1.0))


def _compute_mask(noise, num_remove):
    batch, channels, seq = noise.shape
    return pl.pallas_call(
        functools.partial(_mask_kernel, num_remove=num_remove),
        grid=(batch,),
        in_specs=[pl.BlockSpec((1, channels, seq), lambda b: (b, 0, 0))],
        out_specs=[
            pl.BlockSpec((1, channels, seq), lambda b: (b, 0, 0)),
            pl.BlockSpec((1, channels, seq), lambda b: (b, 0, 0)),
        ],
        out_shape=[
            jax.ShapeDtypeStruct((batch, channels, seq), jnp.float32),
            jax.ShapeDtypeStruct((batch, channels, seq), jnp.float32),
        ],
    )(noise)


def _sc_fill(patch_flat, mult, lo, n_b):
    batch, channels, sf = patch_flat.shape
    seq = mult.shape[-1]
    feat = sf // seq
    mesh = plsc.VectorSubcoreMesh(core_axis_name="core",
                                  subcore_axis_name="subcore")

    @pl.kernel(
        out_type=jax.ShapeDtypeStruct((n_b, channels, sf), jnp.float32),
        mesh=mesh,
    )
    def fill_kernel(mult_hbm, patch_hbm, out_hbm):
        def body(m_vmem, x_vmem, o_vmem):
            @plsc.parallel_loop(0, seq, unroll=4)
            def _row(s):
                m = m_vmem[0, 0, pl.ds(s, 1)][0]
                base = s * feat
                for f in range(0, feat, _SC_LANES):
                    sl = (0, 0, pl.ds(base + f, _SC_LANES))
                    o_vmem[sl] = x_vmem[sl] * m

        pltpu.emit_pipeline(
            body,
            grid=(n_b, channels),
            in_specs=[
                pl.BlockSpec((1, 1, seq), lambda b, c: (b + lo, c, 0)),
                pl.BlockSpec((1, 1, sf), lambda b, c: (b + lo, c, 0)),
            ],
            out_specs=[pl.BlockSpec((1, 1, sf), lambda b, c: (b, c, 0))],
            core_axis_name=("core", "subcore"),
            dimension_semantics=(pltpu.PARALLEL, pltpu.PARALLEL),
        )(mult_hbm, patch_hbm, out_hbm)

    return fill_kernel(mult, patch_flat)


NBUF = 8


def _tc_fill_kernel(mult_ref, patch_hbm, out_hbm, inbuf, obuf, insem, osem,
                    *, n_chunks, lo):
    def in_copy(chunk, slot):
        return pltpu.make_async_copy(
            patch_hbm.at[pl.ds(chunk + lo, 1)], inbuf.at[slot], insem.at[slot])

    def out_copy(chunk, slot):
        return pltpu.make_async_copy(
            obuf.at[slot], out_hbm.at[pl.ds(chunk, 1)], osem.at[slot])

    for s in range(NBUF):
        in_copy(s, s).start()

    def body(c, carry):
        slot = lax.rem(c, NBUF)
        in_copy(c, slot).wait()
        m = mult_ref[c + lo]
        x = inbuf[slot, 0]

        @pl.when(c >= NBUF)
        def _wait_prev_out():
            out_copy(c - NBUF, slot).wait()

        obuf[slot, 0] = x * m[:, :, None]
        out_copy(c, slot).start()

        @pl.when(c + NBUF < n_chunks)
        def _start_next_in():
            in_copy(c + NBUF, slot).start()

        return carry

    lax.fori_loop(0, n_chunks, body, 0)

    for s in range(NBUF):
        chunk = n_chunks - NBUF + s
        out_copy(chunk, chunk % NBUF).wait()


def _tc_fill(patch_input, mult, lo, n_b):
    batch, channels, seq, feat = patch_input.shape
    return pl.pallas_call(
        functools.partial(_tc_fill_kernel, n_chunks=n_b, lo=lo),
        in_specs=[
            pl.BlockSpec(memory_space=pltpu.VMEM),
            pl.BlockSpec(memory_space=pl.ANY),
        ],
        out_specs=pl.BlockSpec(memory_space=pl.ANY),
        out_shape=jax.ShapeDtypeStruct((n_b, channels, seq, feat),
                                       patch_input.dtype),
        scratch_shapes=[
            pltpu.VMEM((NBUF, 1, channels, seq, feat), jnp.float32),
            pltpu.VMEM((NBUF, 1, channels, seq, feat), jnp.float32),
            pltpu.SemaphoreType.DMA((NBUF,)),
            pltpu.SemaphoreType.DMA((NBUF,)),
        ],
    )(mult, patch_input)


def kernel(patch_input, noise):
    batch, channels, seq, feat = patch_input.shape
    len_keep = int(seq * (1 - MASK_RATIO))
    num_remove = seq - len_keep

    mask, mult = _compute_mask(noise, num_remove)
    split = batch // 2
    out_tc = _tc_fill(patch_input, mult, 0, split)
    out_sc = _sc_fill(patch_input.reshape(batch, channels, seq * feat), mult,
                      split, batch - split)
    out = jnp.concatenate(
        [out_tc, out_sc.reshape(batch - split, channels, seq, feat)], axis=0)
    return out, mask.astype(bool)


# speedup vs baseline: 1.5912x; 1.5912x over previous
import functools

import jax
import jax.numpy as jnp
from jax import lax
from jax.experimental import pallas as pl
from jax.experimental.pallas import tpu as pltpu
from jax.experimental.pallas import tpu_sc as plsc

MASK_RATIO = 0.4
MASK_VALUE = 0.0

_SC_LANES = 16


def _mask_kernel(noise_ref, mask_ref, mult_ref, *, num_remove):
    n = noise_ref[0]
    S = n.shape[-1]
    k2 = pltpu.bitcast(n, jnp.int32) * 2
    j_idx = lax.broadcasted_iota(jnp.int32, (1, S, S), 1)
    i_idx = lax.broadcasted_iota(jnp.int32, (1, S, S), 2)
    tri = (j_idx > i_idx).astype(jnp.int32)
    bj = k2[:, :, None] + tri
    greater = bj > k2[:, None, :]
    cnt = jnp.count_nonzero(greater, axis=1).astype(jnp.int32)
    remove = cnt < num_remove
    mask_ref[0] = remove.astype(jnp.float32)
    mult_ref[0] = jnp.where(remove, jnp.float32(MASK_VALUE), jnp.float32(1.0))


def _compute_mask(noise, num_remove):
    batch, channels, seq = noise.shape
    return pl.pallas_call(
        functools.partial(_mask_kernel, num_remove=num_remove),
        grid=(batch,),
        in_specs=[pl.BlockSpec((1, channels, seq), lambda b: (b, 0, 0))],
        out_specs=[
            pl.BlockSpec((1, channels, seq), lambda b: (b, 0, 0)),
            pl.BlockSpec((1, channels, seq), lambda b: (b, 0, 0)),
        ],
        out_shape=[
            jax.ShapeDtypeStruct((batch, channels, seq), jnp.float32),
            jax.ShapeDtypeStruct((batch, channels, seq), jnp.float32),
        ],
    )(noise)


def _sc_fill(patch_flat, mult):
    batch, channels, sf = patch_flat.shape
    seq = mult.shape[-1]
    feat = sf // seq
    mesh = plsc.VectorSubcoreMesh(core_axis_name="core",
                                  subcore_axis_name="subcore")

    @pl.kernel(
        out_type=jax.ShapeDtypeStruct((batch, channels, sf), jnp.float32),
        mesh=mesh,
    )
    def fill_kernel(mult_hbm, patch_hbm, out_hbm):
        def body(m_vmem, x_vmem, o_vmem):
            @plsc.parallel_loop(0, seq, unroll=4)
            def _row(s):
                m = m_vmem[0, 0, pl.ds(s, 1)][0]
                base = s * feat
                for f in range(0, feat, _SC_LANES):
                    sl = (0, 0, pl.ds(base + f, _SC_LANES))
                    o_vmem[sl] = x_vmem[sl] * m

        pltpu.emit_pipeline(
            body,
            grid=(batch, channels),
            in_specs=[
                pl.BlockSpec((1, 1, seq), lambda b, c: (b, c, 0)),
                pl.BlockSpec((1, 1, sf), lambda b, c: (b, c, 0)),
            ],
            out_specs=[pl.BlockSpec((1, 1, sf), lambda b, c: (b, c, 0))],
            core_axis_name=("core", "subcore"),
            dimension_semantics=(pltpu.PARALLEL, pltpu.PARALLEL),
        )(mult_hbm, patch_hbm, out_hbm)

    return fill_kernel(mult, patch_flat)


def kernel(patch_input, noise):
    batch, channels, seq, feat = patch_input.shape
    len_keep = int(seq * (1 - MASK_RATIO))
    num_remove = seq - len_keep

    mask, mult = _compute_mask(noise, num_remove)
    out = _sc_fill(patch_input.reshape(batch, channels, seq * feat), mult)
    return out.reshape(batch, channels, seq, feat), mask.astype(bool)
